# Initial kernel scaffold; baseline (speedup 1.0000x reference)
import jax
import jax.numpy as jnp
from jax.experimental import pallas as pl


def kernel(x, edge_index, edge_attr, edge_index2, edge_attr2, W_ln, W1, W2):
    raise NotImplementedError("write your pallas kernel here")


# speedup vs baseline: 6.7447x; 6.7447x over previous
import functools

import jax
import jax.numpy as jnp
from jax import lax
from jax.experimental import pallas as pl
from jax.experimental.pallas import tpu as pltpu
from jax.experimental.pallas import tpu_sc as plsc

N_NODES = 10000
IN_DIM = 128
OUT_DIM = 128
N_EDGES = 320000

_NS = 16
_CHUNK = 128
_CPS = 8
_SLAB = _CPS * _CHUNK
_NSLAB = 20
_K = _NSLAB * _CPS
_EPT = _K * _CHUNK
_EPAD = _NS * _EPT
_ROWS_PT = 640
_ACC_ROWS = _NS * _ROWS_PT
_LAST_ROWS = N_NODES - (_NS - 1) * _ROWS_PT


def _lane_bcast(v, e):
    idx = jnp.full((16, 1), e, dtype=jnp.int32)
    return lax.gather(
        v, idx,
        lax.GatherDimensionNumbers(
            offset_dims=(), collapsed_slice_dims=(0,), start_index_map=(0,)),
        (1,),
        mode=lax.GatherScatterMode.PROMISE_IN_BOUNDS)


def _mm_body(x_ref, w_ref, o0_ref, o1_ref, o2_ref):
    h = jnp.dot(x_ref[...], w_ref[...], preferred_element_type=jnp.float32)
    o0_ref[...] = h[:, :OUT_DIM]
    o1_ref[...] = h[:, OUT_DIM:2 * OUT_DIM]
    o2_ref[...] = h[:, 2 * OUT_DIM:]


def _matmul3(x, w_cat):
    blk = 1000
    grid = (N_NODES // blk,)
    out = jax.ShapeDtypeStruct((N_NODES, OUT_DIM), jnp.float32)
    return pl.pallas_call(
        _mm_body,
        grid=grid,
        in_specs=[
            pl.BlockSpec((blk, IN_DIM), lambda i: (i, 0)),
            pl.BlockSpec((IN_DIM, 3 * OUT_DIM), lambda i: (0, 0)),
        ],
        out_specs=[pl.BlockSpec((blk, OUT_DIM), lambda i: (i, 0))] * 3,
        out_shape=[out, out, out],
    )(x, w_cat)


def _conv_one_set(sid, src_hbm, dst_hbm, attr_hbm, h_hbm, out_hbm,
                  src_s, dst_s, attr_s, rows_v, acc, isem, gsem):
    def _zrow(i, _):
        for q in range(8):
            rows_v[i, pl.ds(q * 16, 16)] = jnp.zeros((16,), jnp.float32)
        return 0
    lax.fori_loop(0, _CHUNK, _zrow, 0)

    zbase = sid * _ROWS_PT
    for t in range(_ROWS_PT // _CHUNK):
        pltpu.sync_copy(rows_v.at[pl.ds(0, _CHUNK)],
                        acc.at[pl.ds(zbase + t * _CHUNK, _CHUNK)])

    plsc.subcore_barrier()

    def _slab_start(s, b):
        pltpu.async_copy(src_hbm.at[sid, pl.ds(s * _CPS, _CPS)],
                         src_s.at[b], isem.at[b])
        pltpu.async_copy(dst_hbm.at[sid, pl.ds(s * _CPS, _CPS)],
                         dst_s.at[b], isem.at[b])
        pltpu.async_copy(attr_hbm.at[sid, 0, pl.ds(s * _SLAB, _SLAB)],
                         attr_s.at[b], isem.at[b])

    def _slab_wait(s, b):
        pltpu.make_async_copy(src_hbm.at[sid, pl.ds(s * _CPS, _CPS)],
                              src_s.at[b], isem.at[b]).wait()
        pltpu.make_async_copy(dst_hbm.at[sid, pl.ds(s * _CPS, _CPS)],
                              dst_s.at[b], isem.at[b]).wait()
        pltpu.make_async_copy(attr_hbm.at[sid, 0, pl.ds(s * _SLAB, _SLAB)],
                              attr_s.at[b], isem.at[b]).wait()

    def _process_slab(b):
        def _chunk(c, _):
            pltpu.async_copy(h_hbm.at[src_s.at[b, c]], rows_v, gsem).wait()

            def _grp(g, _):
                a16 = attr_s[b, pl.ds(c * _CHUNK + g * 16, 16)]
                for e in range(16):
                    ae = _lane_bcast(a16, e)
                    r = g * 16 + e
                    for q in range(8):
                        sl = pl.ds(q * 16, 16)
                        rows_v[r, sl] = rows_v[r, sl] * ae
                return 0
            lax.fori_loop(0, 8, _grp, 0)

            pltpu.sync_copy(rows_v, acc.at[dst_s.at[b, c]], add=True)
            return 0
        lax.fori_loop(0, _CPS, _chunk, 0)

    _slab_start(0, 0)

    def _pair(t, _):
        _slab_start(2 * t + 1, 1)
        _slab_wait(2 * t, 0)
        _process_slab(0)

        @pl.when(t < _NSLAB // 2 - 1)
        def _():
            _slab_start(2 * t + 2, 0)
        _slab_wait(2 * t + 1, 1)
        _process_slab(1)
        return 0
    lax.fori_loop(0, _NSLAB // 2, _pair, 0)

    plsc.subcore_barrier()

    @pl.when(sid < _NS - 1)
    def _():
        pltpu.sync_copy(acc.at[pl.ds(zbase, _ROWS_PT)],
                        out_hbm.at[pl.ds(zbase, _ROWS_PT)])

    @pl.when(sid == _NS - 1)
    def _():
        pltpu.sync_copy(acc.at[pl.ds(zbase, _LAST_ROWS)],
                        out_hbm.at[pl.ds(zbase, _LAST_ROWS)])


def _sc_body(src1, dst1, attr1, h1, src2, dst2, attr2, h2, o1, o2,
             src_s, dst_s, attr_s, rows_v, acc, isem, gsem):
    cid = lax.axis_index("c")
    sid = lax.axis_index("s")

    @pl.when(cid == 0)
    def _():
        _conv_one_set(sid, src1, dst1, attr1, h1, o1,
                      src_s, dst_s, attr_s, rows_v, acc, isem, gsem)

    @pl.when(cid == 1)
    def _():
        _conv_one_set(sid, src2, dst2, attr2, h2, o2,
                      src_s, dst_s, attr_s, rows_v, acc, isem, gsem)


_sc_conv = functools.partial(
    pl.kernel,
    out_type=(jax.ShapeDtypeStruct((N_NODES, OUT_DIM), jnp.float32),
              jax.ShapeDtypeStruct((N_NODES, OUT_DIM), jnp.float32)),
    mesh=plsc.VectorSubcoreMesh(core_axis_name="c", subcore_axis_name="s"),
    scratch_types=[
        pltpu.VMEM((2, _CPS, _CHUNK), jnp.int32),
        pltpu.VMEM((2, _CPS, _CHUNK), jnp.int32),
        pltpu.VMEM((2, _SLAB), jnp.float32),
        pltpu.VMEM((_CHUNK, OUT_DIM), jnp.float32),
        pltpu.VMEM_SHARED((_ACC_ROWS, OUT_DIM), jnp.float32),
        pltpu.SemaphoreType.DMA((2,)),
        pltpu.SemaphoreType.DMA,
    ],
)(_sc_body)


def _prep_edges(edge_index, edge_attr):
    pad = _EPAD - N_EDGES
    spread = (jnp.arange(pad, dtype=jnp.int32) * 97) % N_NODES
    src = jnp.concatenate([edge_index[0].astype(jnp.int32), spread])
    dst = jnp.concatenate([edge_index[1].astype(jnp.int32), spread])
    attr = jnp.concatenate([edge_attr, jnp.zeros((pad,), jnp.float32)])
    return (src.reshape(_NS, _K, _CHUNK), dst.reshape(_NS, _K, _CHUNK),
            attr.reshape(_NS, 1, _EPT))


@jax.jit
def kernel(x, edge_index, edge_attr, edge_index2, edge_attr2, W_ln, W1, W2):
    w_cat = jnp.concatenate([W_ln, W1, W2], axis=1)
    x0, h1, h2 = _matmul3(x, w_cat)
    src1, dst1, attr1 = _prep_edges(edge_index, edge_attr)
    src2, dst2, attr2 = _prep_edges(edge_index2, edge_attr2)
    o1, o2 = _sc_conv(src1, dst1, attr1, h1, src2, dst2, attr2, h2)
    return x0, o1, o2


# speedup vs baseline: 9.9641x; 1.4773x over previous
import functools

import jax
import jax.numpy as jnp
from jax import lax
from jax.experimental import pallas as pl
from jax.experimental.pallas import tpu as pltpu
from jax.experimental.pallas import tpu_sc as plsc

N_NODES = 10000
IN_DIM = 128
OUT_DIM = 128
N_EDGES = 320000

_NS = 16
_CHUNK = 128
_CPS = 8
_SLAB = _CPS * _CHUNK
_NSLAB = 20
_K = _NSLAB * _CPS
_EPT = _K * _CHUNK
_EPAD = _NS * _EPT
_ROWS_PT = 640
_ACC_ROWS = _NS * _ROWS_PT
_LAST_ROWS = N_NODES - (_NS - 1) * _ROWS_PT


def _lane_bcast(v, e):
    idx = jnp.full((16, 1), e, dtype=jnp.int32)
    return lax.gather(
        v, idx,
        lax.GatherDimensionNumbers(
            offset_dims=(), collapsed_slice_dims=(0,), start_index_map=(0,)),
        (1,),
        mode=lax.GatherScatterMode.PROMISE_IN_BOUNDS)


def _mm_body(x_ref, w_ref, o0_ref, o1_ref, o2_ref):
    h = jnp.dot(x_ref[...], w_ref[...], preferred_element_type=jnp.float32)
    o0_ref[...] = h[:, :OUT_DIM]
    o1_ref[...] = h[:, OUT_DIM:2 * OUT_DIM]
    o2_ref[...] = h[:, 2 * OUT_DIM:]


def _matmul3(x, w_cat):
    blk = 1000
    grid = (N_NODES // blk,)
    out = jax.ShapeDtypeStruct((N_NODES, OUT_DIM), jnp.float32)
    return pl.pallas_call(
        _mm_body,
        grid=grid,
        in_specs=[
            pl.BlockSpec((blk, IN_DIM), lambda i: (i, 0)),
            pl.BlockSpec((IN_DIM, 3 * OUT_DIM), lambda i: (0, 0)),
        ],
        out_specs=[pl.BlockSpec((blk, OUT_DIM), lambda i: (i, 0))] * 3,
        out_shape=[out, out, out],
    )(x, w_cat)


def _conv_one_set(sid, src_hbm, dst_hbm, attr_hbm, h_hbm, out_hbm,
                  src_s, dst_s, attr_s, rows_v, acc, isem, gsem):
    def _zrow(i, _):
        for q in range(8):
            rows_v[0, i, pl.ds(q * 16, 16)] = jnp.zeros((16,), jnp.float32)
        return 0
    lax.fori_loop(0, _CHUNK, _zrow, 0)

    zbase = sid * _ROWS_PT
    for t in range(_ROWS_PT // _CHUNK):
        pltpu.sync_copy(rows_v.at[0],
                        acc.at[pl.ds(zbase + t * _CHUNK, _CHUNK)])

    plsc.subcore_barrier()

    def _slab_start(s, b):
        pltpu.async_copy(src_hbm.at[sid, pl.ds(s * _CPS, _CPS)],
                         src_s.at[b], isem.at[b])
        pltpu.async_copy(dst_hbm.at[sid, pl.ds(s * _CPS, _CPS)],
                         dst_s.at[b], isem.at[b])
        pltpu.async_copy(attr_hbm.at[sid, 0, pl.ds(s * _SLAB, _SLAB)],
                         attr_s.at[b], isem.at[b])

    def _slab_wait(s, b):
        pltpu.make_async_copy(src_hbm.at[sid, pl.ds(s * _CPS, _CPS)],
                              src_s.at[b], isem.at[b]).wait()
        pltpu.make_async_copy(dst_hbm.at[sid, pl.ds(s * _CPS, _CPS)],
                              dst_s.at[b], isem.at[b]).wait()
        pltpu.make_async_copy(attr_hbm.at[sid, 0, pl.ds(s * _SLAB, _SLAB)],
                              attr_s.at[b], isem.at[b]).wait()

    def _g_start(b, c, rb):
        pltpu.async_copy(h_hbm.at[src_s.at[b, c]], rows_v.at[rb],
                         gsem.at[rb])

    def _g_wait(b, c, rb):
        pltpu.make_async_copy(h_hbm.at[src_s.at[b, c]], rows_v.at[rb],
                              gsem.at[rb]).wait()

    def _scale(b, c, rb):
        def _grp(g, _):
            a16 = attr_s[b, pl.ds(c * _CHUNK + g * 16, 16)]
            for e in range(16):
                ae = _lane_bcast(a16, e)
                r = g * 16 + e
                for q in range(8):
                    sl = pl.ds(q * 16, 16)
                    rows_v[rb, r, sl] = rows_v[rb, r, sl] * ae
            return 0
        lax.fori_loop(0, 8, _grp, 0)

    def _process_slab(b):
        _g_start(b, 0, 0)

        def _cpair(c2, _):
            _g_start(b, 2 * c2 + 1, 1)
            _g_wait(b, 2 * c2, 0)
            _scale(b, 2 * c2, 0)
            pltpu.sync_copy(rows_v.at[0], acc.at[dst_s.at[b, 2 * c2]],
                            add=True)

            @pl.when(c2 < _CPS // 2 - 1)
            def _():
                _g_start(b, 2 * c2 + 2, 0)
            _g_wait(b, 2 * c2 + 1, 1)
            _scale(b, 2 * c2 + 1, 1)
            pltpu.sync_copy(rows_v.at[1], acc.at[dst_s.at[b, 2 * c2 + 1]],
                            add=True)
            return 0
        lax.fori_loop(0, _CPS // 2, _cpair, 0)

    _slab_start(0, 0)

    def _pair(t, _):
        _slab_start(2 * t + 1, 1)
        _slab_wait(2 * t, 0)
        _process_slab(0)

        @pl.when(t < _NSLAB // 2 - 1)
        def _():
            _slab_start(2 * t + 2, 0)
        _slab_wait(2 * t + 1, 1)
        _process_slab(1)
        return 0
    lax.fori_loop(0, _NSLAB // 2, _pair, 0)

    plsc.subcore_barrier()

    @pl.when(sid < _NS - 1)
    def _():
        pltpu.sync_copy(acc.at[pl.ds(zbase, _ROWS_PT)],
                        out_hbm.at[pl.ds(zbase, _ROWS_PT)])

    @pl.when(sid == _NS - 1)
    def _():
        pltpu.sync_copy(acc.at[pl.ds(zbase, _LAST_ROWS)],
                        out_hbm.at[pl.ds(zbase, _LAST_ROWS)])


def _sc_body(src1, dst1, attr1, h1, src2, dst2, attr2, h2, o1, o2,
             src_s, dst_s, attr_s, rows_v, acc, isem, gsem):
    cid = lax.axis_index("c")
    sid = lax.axis_index("s")

    @pl.when(cid == 0)
    def _():
        _conv_one_set(sid, src1, dst1, attr1, h1, o1,
                      src_s, dst_s, attr_s, rows_v, acc, isem, gsem)

    @pl.when(cid == 1)
    def _():
        _conv_one_set(sid, src2, dst2, attr2, h2, o2,
                      src_s, dst_s, attr_s, rows_v, acc, isem, gsem)


_sc_conv = functools.partial(
    pl.kernel,
    out_type=(jax.ShapeDtypeStruct((N_NODES, OUT_DIM), jnp.float32),
              jax.ShapeDtypeStruct((N_NODES, OUT_DIM), jnp.float32)),
    mesh=plsc.VectorSubcoreMesh(core_axis_name="c", subcore_axis_name="s"),
    scratch_types=[
        pltpu.VMEM((2, _CPS, _CHUNK), jnp.int32),
        pltpu.VMEM((2, _CPS, _CHUNK), jnp.int32),
        pltpu.VMEM((2, _SLAB), jnp.float32),
        pltpu.VMEM((2, _CHUNK, OUT_DIM), jnp.float32),
        pltpu.VMEM_SHARED((_ACC_ROWS, OUT_DIM), jnp.float32),
        pltpu.SemaphoreType.DMA((2,)),
        pltpu.SemaphoreType.DMA((2,)),
    ],
)(_sc_body)


def _prep_edges(edge_index, edge_attr):
    pad = _EPAD - N_EDGES
    spread = (jnp.arange(pad, dtype=jnp.int32) * 97) % N_NODES
    src = jnp.concatenate([edge_index[0].astype(jnp.int32), spread])
    dst = jnp.concatenate([edge_index[1].astype(jnp.int32), spread])
    attr = jnp.concatenate([edge_attr, jnp.zeros((pad,), jnp.float32)])
    return (src.reshape(_NS, _K, _CHUNK), dst.reshape(_NS, _K, _CHUNK),
            attr.reshape(_NS, 1, _EPT))


@jax.jit
def kernel(x, edge_index, edge_attr, edge_index2, edge_attr2, W_ln, W1, W2):
    w_cat = jnp.concatenate([W_ln, W1, W2], axis=1)
    x0, h1, h2 = _matmul3(x, w_cat)
    src1, dst1, attr1 = _prep_edges(edge_index, edge_attr)
    src2, dst2, attr2 = _prep_edges(edge_index2, edge_attr2)
    o1, o2 = _sc_conv(src1, dst1, attr1, h1, src2, dst2, attr2, h2)
    return x0, o1, o2
